# baseline (speedup 1.0000x reference)
import functools

import jax
import jax.numpy as jnp
from jax import lax
from jax.experimental import pallas as pl
from jax.experimental.pallas import tpu as pltpu
from jax.experimental.pallas import tpu_sc as plsc

EMB = 16
GROUP = 128
NC = 2
NS = 16
NW = NC * NS
PAD = 0


def _emb_body(ng, idx_hbm, table_hbm, out_hbm, idx_v, rows_v, sem_g, sem_s):
    wid = lax.axis_index("s") * NC + lax.axis_index("c")
    gbase = wid * ng
    pltpu.sync_copy(idx_hbm.at[pl.ds(gbase, ng)], idx_v)

    iota = lax.iota(jnp.int32, 16)
    zeros = jnp.zeros((16,), jnp.float32)

    def fix_group(j):
        m = None
        for g in range(GROUP // 16):
            iv = idx_v[j, pl.ds(g * 16, 16)]
            m = iv if m is None else jnp.minimum(m, iv)
        cnt = plsc.all_reduce_population_count(m == PAD)

        @pl.when(cnt[0] > 0)
        def _():
            for g in range(GROUP // 16):
                msk = idx_v[j, pl.ds(g * 16, 16)] == PAD
                rows = iota + g * 16
                for c in range(EMB):
                    plsc.store_scatter(
                        rows_v, [rows, jnp.full((16,), c, jnp.int32)],
                        zeros, mask=msk)

    def body(j, carry):
        pltpu.async_copy(table_hbm.at[idx_v.at[j]], rows_v, sem_g).wait()
        fix_group(j)
        pltpu.async_copy(
            rows_v, out_hbm.at[pl.ds((gbase + j) * GROUP, GROUP)], sem_s
        ).wait()
        return carry

    lax.fori_loop(0, ng, body, 0)


def kernel(input, weight):
    ids = input.astype(jnp.int32)
    b, s = ids.shape
    tot = b * s
    ngroups = tot // GROUP
    ng = ngroups // NW
    idx2d = ids.reshape(ngroups, GROUP)

    mesh = plsc.VectorSubcoreMesh(core_axis_name="c", subcore_axis_name="s")
    run = pl.kernel(
        functools.partial(_emb_body, ng),
        mesh=mesh,
        compiler_params=pltpu.CompilerParams(
            use_tc_tiling_on_sc=False, needs_layout_passes=False),
        out_type=jax.ShapeDtypeStruct((tot, EMB), jnp.float32),
        scratch_types=[
            pltpu.VMEM((ng, GROUP), jnp.int32),
            pltpu.VMEM((GROUP, EMB), jnp.float32),
            pltpu.SemaphoreType.DMA,
            pltpu.SemaphoreType.DMA,
        ],
    )
    out = run(idx2d, weight)
    return out.reshape(b, s, EMB)


# speedup vs baseline: 1.1465x; 1.1465x over previous
import functools

import jax
import jax.numpy as jnp
from jax import lax
from jax.experimental import pallas as pl
from jax.experimental.pallas import tpu as pltpu
from jax.experimental.pallas import tpu_sc as plsc

EMB = 16
GROUP = 128
GPS = 4
SB = GROUP * GPS
NC = 2
NS = 16
NW = NC * NS
PAD = 0


def _emb_body(ng, idx_hbm, table_hbm, out_hbm, idx_v, rows_v,
              sg0, sg1, ss0, ss1):
    nt = ng // GPS
    wid = lax.axis_index("s") * NC + lax.axis_index("c")
    rbase = wid * ng * GROUP
    pltpu.sync_copy(idx_hbm.at[pl.ds(rbase, ng * GROUP)], idx_v)

    iota = lax.iota(jnp.int32, 16)
    zeros = jnp.zeros((16,), jnp.float32)
    sg = (sg0, sg1)
    ss = (ss0, ss1)

    def gathers(t, p):
        return pltpu.make_async_copy(
            table_hbm.at[idx_v.at[pl.ds(t * SB, SB)]],
            rows_v.at[pl.ds(p * SB, SB)], sg[p])

    def store(t, p):
        return pltpu.make_async_copy(
            rows_v.at[pl.ds(p * SB, SB)],
            out_hbm.at[pl.ds(rbase + t * SB, SB)], ss[p])

    def fix(t, p):
        m = None
        for g in range(SB // 16):
            iv = idx_v[pl.ds(t * SB + g * 16, 16)]
            m = iv if m is None else jnp.minimum(m, iv)
        cnt = plsc.all_reduce_population_count(m == PAD)

        @pl.when(cnt[0] > 0)
        def _():
            for g in range(SB // 16):
                msk = idx_v[pl.ds(t * SB + g * 16, 16)] == PAD
                rows = iota + (p * SB + g * 16)
                for c in range(EMB):
                    plsc.store_scatter(
                        rows_v, [rows, jnp.full((16,), c, jnp.int32)],
                        zeros, mask=msk)

    gathers(0, 0).start()

    def body(t2, carry):
        for p_static in range(2):
            t = t2 * 2 + p_static
            p = p_static

            @pl.when(t + 1 < nt)
            def _():
                @pl.when(t >= 1)
                def _():
                    store(t - 1, 1 - p).wait()
                gathers(t + 1, 1 - p).start()

            gathers(t, p).wait()
            fix(t, p)
            store(t, p).start()
        return carry

    lax.fori_loop(0, nt // 2, body, 0)
    store(nt - 2, 0).wait()
    store(nt - 1, 1).wait()


def kernel(input, weight):
    ids = input.astype(jnp.int32)
    b, s = ids.shape
    tot = b * s
    ngroups = tot // GROUP
    ng = ngroups // NW
    idx_flat = ids.reshape(tot)

    mesh = plsc.VectorSubcoreMesh(core_axis_name="c", subcore_axis_name="s")
    run = pl.kernel(
        functools.partial(_emb_body, ng),
        mesh=mesh,
        compiler_params=pltpu.CompilerParams(
            use_tc_tiling_on_sc=False, needs_layout_passes=False),
        out_type=jax.ShapeDtypeStruct((tot, EMB), jnp.float32),
        scratch_types=[
            pltpu.VMEM((ng * GROUP,), jnp.int32),
            pltpu.VMEM((2 * SB, EMB), jnp.float32),
            pltpu.SemaphoreType.DMA,
            pltpu.SemaphoreType.DMA,
            pltpu.SemaphoreType.DMA,
            pltpu.SemaphoreType.DMA,
        ],
    )
    out = run(idx_flat, weight)
    return out.reshape(b, s, EMB)


# speedup vs baseline: 1.5262x; 1.3312x over previous
import functools

import jax
import jax.numpy as jnp
from jax import lax
from jax.experimental import pallas as pl
from jax.experimental.pallas import tpu as pltpu
from jax.experimental.pallas import tpu_sc as plsc

EMB = 16
BW = 128
SS = 4
NC = 2
NS = 16
NW = NC * NS
PAD = 0


def _emb_body(seq, ids_t_hbm, table_hbm, out_hbm, idx_v, rows_v, tp_v,
              sg0, sg1, ss0, ss1):
    nt = seq // SS
    wid = lax.axis_index("s") * NC + lax.axis_index("c")
    b0 = wid * BW
    pltpu.sync_copy(ids_t_hbm.at[:, pl.ds(b0, BW)], idx_v)

    iota = lax.iota(jnp.int32, 16)
    zeros = jnp.zeros((16,), jnp.float32)
    sg = (sg0, sg1)
    ss = (ss0, ss1)

    def gather_q(t, p, q):
        return pltpu.make_async_copy(
            table_hbm.at[idx_v.at[t * SS + q]],
            rows_v.at[pl.ds((p * SS + q) * BW, BW)], sg[p])

    def write_qe(t, p, q, eb):
        return pltpu.make_async_copy(
            tp_v.at[p, q, pl.ds(eb * 8, 8)],
            out_hbm.at[t * SS + q, eb, wid], ss[p])

    def transpose_q(t, p, q):
        base = (p * SS + q) * BW
        for k in range(BW // 16):
            msk = idx_v[t * SS + q, pl.ds(k * 16, 16)] == PAD
            rowv = iota + (base + k * 16)
            for e in range(EMB):
                val = plsc.load_gather(
                    rows_v, [rowv, jnp.full((16,), e, jnp.int32)])
                tp_v[p, q, e, pl.ds(k * 16, 16)] = jnp.where(msk, 0.0, val)

    for q in range(SS):
        gather_q(0, 0, q).start()

    def body(t2, carry):
        for p in range(2):
            t = t2 * 2 + p

            @pl.when(t < nt)
            def _():
                @pl.when(t + 1 < nt)
                def _():
                    @pl.when(t >= 1)
                    def _():
                        for q in range(SS):
                            for eb in range(2):
                                write_qe(t - 1, 1 - p, q, eb).wait()
                    for q in range(SS):
                        gather_q(t + 1, 1 - p, q).start()

                for q in range(SS):
                    gather_q(t, p, q).wait()
                for q in range(SS):
                    transpose_q(t, p, q)
                for q in range(SS):
                    for eb in range(2):
                        write_qe(t, p, q, eb).start()
        return carry

    lax.fori_loop(0, (nt + 2) // 2, body, 0)
    for q in range(SS):
        for eb in range(2):
            write_qe(nt - 2, (nt - 2) % 2, q, eb).wait()
            write_qe(nt - 1, (nt - 1) % 2, q, eb).wait()


def kernel(input, weight):
    ids = input.astype(jnp.int32)
    b, seq = ids.shape
    ids_t = ids.T
    nb = b // BW

    mesh = plsc.VectorSubcoreMesh(core_axis_name="c", subcore_axis_name="s")
    run = pl.kernel(
        functools.partial(_emb_body, seq),
        mesh=mesh,
        compiler_params=pltpu.CompilerParams(
            use_tc_tiling_on_sc=False, needs_layout_passes=False),
        out_type=jax.ShapeDtypeStruct((seq, EMB // 8, nb, 8, BW),
                                      jnp.float32),
        scratch_types=[
            pltpu.VMEM((seq, BW), jnp.int32),
            pltpu.VMEM((2 * SS * BW, EMB), jnp.float32),
            pltpu.VMEM((2, SS, EMB, BW), jnp.float32),
            pltpu.SemaphoreType.DMA,
            pltpu.SemaphoreType.DMA,
            pltpu.SemaphoreType.DMA,
            pltpu.SemaphoreType.DMA,
        ],
    )
    out_phys = run(ids_t, weight)
    return out_phys.transpose(2, 4, 0, 1, 3).reshape(b, seq, EMB)


# speedup vs baseline: 1.8918x; 1.2396x over previous
import functools

import jax
import jax.numpy as jnp
from jax import lax
from jax.experimental import pallas as pl
from jax.experimental.pallas import tpu as pltpu
from jax.experimental.pallas import tpu_sc as plsc

EMB = 16
BW = 128
SS = 4
NC = 2
NS = 16
NW = NC * NS
PAD = 0


def _emb_body(seq, ids_t_hbm, table_hbm, out_hbm, idx_v, rows_v, tp_v,
              sg0, sg1, ss0, ss1):
    nt = seq // SS
    wid = lax.axis_index("s") * NC + lax.axis_index("c")
    b0 = wid * BW
    pltpu.sync_copy(ids_t_hbm.at[:, pl.ds(b0, BW)], idx_v)

    iota = lax.iota(jnp.int32, 16)
    zeros = jnp.zeros((16,), jnp.float32)
    sg = (sg0, sg1)
    ss = (ss0, ss1)

    def gather_q(t, p, q):
        return pltpu.make_async_copy(
            table_hbm.at[idx_v.at[t * SS + q]],
            rows_v.at[pl.ds((p * SS + q) * BW, BW)], sg[p])

    def write_qe(t, p, q, eb):
        return pltpu.make_async_copy(
            tp_v.at[p, q, pl.ds(eb * 8, 8)],
            out_hbm.at[t * SS + q, eb, wid], ss[p])

    def fix(t, p):
        m = None
        for g in range(SS * BW // 16):
            iv = idx_v[t * SS + g // (BW // 16),
                       pl.ds((g % (BW // 16)) * 16, 16)]
            m = iv if m is None else jnp.minimum(m, iv)
        cnt = plsc.all_reduce_population_count(m == PAD)

        @pl.when(cnt[0] > 0)
        def _():
            for q in range(SS):
                base = (p * SS + q) * BW
                for k in range(BW // 16):
                    msk = idx_v[t * SS + q, pl.ds(k * 16, 16)] == PAD
                    rows = iota + (base + k * 16)
                    for c in range(EMB):
                        plsc.store_scatter(
                            rows_v, [rows, jnp.full((16,), c, jnp.int32)],
                            zeros, mask=msk)

    def transpose_q(t, p, q):
        base = (p * SS + q) * BW
        for k in range(BW // 16):
            rowv = iota + (base + k * 16)
            vals = [
                plsc.load_gather(rows_v, [rowv, jnp.full((16,), e, jnp.int32)])
                for e in range(EMB)
            ]
            for e in range(EMB):
                tp_v[p, q, e, pl.ds(k * 16, 16)] = vals[e]

    for q in range(SS):
        gather_q(0, 0, q).start()

    def body(t2, carry):
        for p in range(2):
            t = t2 * 2 + p

            @pl.when(t < nt)
            def _():
                @pl.when(t + 1 < nt)
                def _():
                    @pl.when(t >= 1)
                    def _():
                        for q in range(SS):
                            for eb in range(2):
                                write_qe(t - 1, 1 - p, q, eb).wait()
                    for q in range(SS):
                        gather_q(t + 1, 1 - p, q).start()

                for q in range(SS):
                    gather_q(t, p, q).wait()
                fix(t, p)
                for q in range(SS):
                    transpose_q(t, p, q)
                for q in range(SS):
                    for eb in range(2):
                        write_qe(t, p, q, eb).start()
        return carry

    lax.fori_loop(0, (nt + 2) // 2, body, 0)
    for q in range(SS):
        for eb in range(2):
            write_qe(nt - 2, (nt - 2) % 2, q, eb).wait()
            write_qe(nt - 1, (nt - 1) % 2, q, eb).wait()


def kernel(input, weight):
    ids = input.astype(jnp.int32)
    b, seq = ids.shape
    ids_t = ids.T
    nb = b // BW

    mesh = plsc.VectorSubcoreMesh(core_axis_name="c", subcore_axis_name="s")
    run = pl.kernel(
        functools.partial(_emb_body, seq),
        mesh=mesh,
        compiler_params=pltpu.CompilerParams(
            use_tc_tiling_on_sc=False, needs_layout_passes=False),
        out_type=jax.ShapeDtypeStruct((seq, EMB // 8, nb, 8, BW),
                                      jnp.float32),
        scratch_types=[
            pltpu.VMEM((seq, BW), jnp.int32),
            pltpu.VMEM((2 * SS * BW, EMB), jnp.float32),
            pltpu.VMEM((2, SS, EMB, BW), jnp.float32),
            pltpu.SemaphoreType.DMA,
            pltpu.SemaphoreType.DMA,
            pltpu.SemaphoreType.DMA,
            pltpu.SemaphoreType.DMA,
        ],
    )
    out_phys = run(ids_t, weight)
    return out_phys.transpose(2, 4, 0, 1, 3).reshape(b, seq, EMB)
